# baseline (speedup 1.0000x reference)
import functools
import jax
import jax.numpy as jnp
from jax import lax
from jax.experimental import pallas as pl
from jax.experimental.pallas import tpu as pltpu
from jax.experimental.pallas import tpu_sc as plsc

N_G = 978
F = 64
H = 2
FH = 128
B = 16
D_FF = 100
NEG = 0.2
GB = 163
N_STEPS = N_G // GB


def _proj_body(x_ref, w_ref, b_ref, o_ref):
    o_ref[0] = (
        jnp.dot(x_ref[0], w_ref[...], preferred_element_type=jnp.float32)
        + b_ref[...]
    )


@jax.jit
def _proj(x, w_cat, b_cat):
    return pl.pallas_call(
        _proj_body,
        grid=(B,),
        in_specs=[
            pl.BlockSpec((1, N_G, F), lambda b: (b, 0, 0)),
            pl.BlockSpec((F, 2 * FH), lambda b: (0, 0)),
            pl.BlockSpec((1, 2 * FH), lambda b: (0, 0)),
        ],
        out_specs=pl.BlockSpec((1, N_G, 2 * FH), lambda b: (b, 0, 0)),
        out_shape=jax.ShapeDtypeStruct((B, N_G, 2 * FH), jnp.float32),
    )(x, w_cat, b_cat)


def _ln(x, w, b):
    mu = jnp.mean(x, axis=-1, keepdims=True)
    xc = x - mu
    var = jnp.mean(xc * xc, axis=-1, keepdims=True)
    return xc * jax.lax.rsqrt(var + 1e-5) * w + b


def _dense_body(xr_ref, den_ref, xin_ref, gatb_ref, linw_ref, linb_ref,
                f1w_ref, f1b_ref, f2w_ref, f2b_ref, ln1w_ref, ln1b_ref,
                ln2w_ref, ln2b_ref, o_ref):
    gatb = gatb_ref[...]
    ln1w = ln1w_ref[...]
    ln1b = ln1b_ref[...]
    ln2w = ln2w_ref[...]
    ln2b = ln2b_ref[...]

    def body(g, carry):
        xs = xr_ref[pl.ds(g, 1)][0]
        den = den_ref[pl.ds(g, 1)][0]
        d0 = den[:, 0:1] + 1e-16
        d1 = den[:, 1:2] + 1e-16
        xc = 0.5 * (xs[:, :F] / d0 + xs[:, F:FH] / d1) + gatb
        y = lax.dot_general(
            xc, linw_ref[pl.ds(g, 1)][0], (((1,), (1,)), ((), ())),
            preferred_element_type=jnp.float32) + linb_ref[0, pl.ds(g, 1)]
        x1 = _ln(xin_ref[pl.ds(g, 1)][0] + y, ln1w, ln1b)
        h = jnp.maximum(
            lax.dot_general(
                x1, f1w_ref[pl.ds(g, 1)][0], (((1,), (1,)), ((), ())),
                preferred_element_type=jnp.float32) + f1b_ref[0, pl.ds(g, 1)],
            0.0)
        y2 = lax.dot_general(
            h, f2w_ref[pl.ds(g, 1)][0], (((1,), (1,)), ((), ())),
            preferred_element_type=jnp.float32) + f2b_ref[0, pl.ds(g, 1)]
        o_ref[pl.ds(g, 1)] = _ln(x1 + y2, ln2w, ln2b)[None]
        return carry

    lax.fori_loop(0, GB, body, 0)


@jax.jit
def _dense_block(x_gat_t, den_t, x_in_t, gat_bias, lin_w, lin_b, ff1_w,
                 ff1_b, ff2_w, ff2_b, ln1_w, ln1_b, ln2_w, ln2_b):
    gb2 = lambda shape: pl.BlockSpec(shape, lambda i: (0, 0))
    return pl.pallas_call(
        _dense_body,
        grid=(N_STEPS,),
        in_specs=[
            pl.BlockSpec((GB, B, FH), lambda i: (i, 0, 0)),
            pl.BlockSpec((GB, B, 2), lambda i: (i, 0, 0)),
            pl.BlockSpec((GB, B, F), lambda i: (i, 0, 0)),
            gb2((1, F)),
            pl.BlockSpec((GB, F, F), lambda i: (i, 0, 0)),
            pl.BlockSpec((1, GB, F), lambda i: (i, 0, 0)),
            pl.BlockSpec((GB, D_FF, F), lambda i: (i, 0, 0)),
            pl.BlockSpec((1, GB, D_FF), lambda i: (i, 0, 0)),
            pl.BlockSpec((GB, F, D_FF), lambda i: (i, 0, 0)),
            pl.BlockSpec((1, GB, F), lambda i: (i, 0, 0)),
            gb2((1, F)),
            gb2((1, F)),
            gb2((1, F)),
            gb2((1, F)),
        ],
        out_specs=pl.BlockSpec((GB, B, F), lambda i: (i, 0, 0)),
        out_shape=jax.ShapeDtypeStruct((N_G, B, F), jnp.float32),
    )(x_gat_t, den_t, x_in_t, gat_bias.reshape(1, F), lin_w,
      lin_b.reshape(N_STEPS, GB, F), ff1_w, ff1_b.reshape(N_STEPS, GB, D_FF),
      ff2_w, ff2_b.reshape(N_STEPS, GB, F), ln1_w.reshape(1, F),
      ln1_b.reshape(1, F), ln2_w.reshape(1, F), ln2_b.reshape(1, F))


N_TILES = 16
CW = 96
NCH = 11
EPT = CW * NCH
E_PAD = EPT * N_TILES
ACC_R = 1024
BPC = B // 2


def _sc_gat_body(xl_hbm, xr_hbm, src_hbm, dstg_hbm, dst2_hbm, att_hbm,
                 zeros_hbm, out_hbm, dout_hbm, src_v, dstg_v, gsrc, gdst,
                 dst2_v, att_v, xi, xj, wbuf, dbuf, a0, a1, acc, dacc, sem):
    c = lax.axis_index("c")
    s = lax.axis_index("s")
    r0 = s * (ACC_R // N_TILES)

    pltpu.sync_copy(src_hbm.at[s], src_v)
    pltpu.sync_copy(dstg_hbm.at[s], dstg_v)
    pltpu.sync_copy(dst2_hbm.at[s], dst2_v)
    pltpu.sync_copy(att_hbm, att_v)
    att_vals = [att_v[f] for f in range(8)]
    lane = lax.iota(jnp.int32, 16)

    def lane_sum(v):
        for m in (8, 4, 2, 1):
            v = v + v.at[lane ^ m].get(mode="promise_in_bounds")
        return v

    pltpu.sync_copy(zeros_hbm.at[pl.ds(0, CW)], dbuf)

    def batch_body(bb, carry):
        b = c * BPC + bb
        pltpu.sync_copy(zeros_hbm.at[pl.ds(r0, ACC_R // N_TILES)],
                        acc.at[pl.ds(r0, ACC_R // N_TILES)])
        pltpu.sync_copy(zeros_hbm.at[pl.ds(r0, ACC_R // N_TILES)],
                        dacc.at[pl.ds(r0, ACC_R // N_TILES)])
        off = b * N_G

        def idx_body(k, carry2):
            sl = pl.ds(k * 16, 16)
            gsrc[sl] = src_v[sl] + off
            gdst[sl] = dstg_v[sl] + off
            return carry2

        lax.fori_loop(0, EPT // 16, idx_body, 0)
        plsc.subcore_barrier()

        def chunk_body(j, carry3):
            e0 = j * CW
            cp1 = pltpu.async_copy(xl_hbm.at[gsrc.at[pl.ds(e0, CW)]], xj, sem)
            cp2 = pltpu.async_copy(xr_hbm.at[gdst.at[pl.ds(e0, CW)]], xi, sem)
            cp1.wait()
            cp2.wait()

            def alpha_body(g, carry4):
                av0 = jnp.zeros((16,), jnp.float32)
                av1 = jnp.zeros((16,), jnp.float32)
                for k in range(16):
                    e = g * 16 + k
                    s0 = jnp.zeros((16,), jnp.float32)
                    s1 = jnp.zeros((16,), jnp.float32)
                    for f in range(4):
                        t = (xi[e, pl.ds(f * 16, 16)]
                             + xj[e, pl.ds(f * 16, 16)])
                        lr = 0.6 * t + 0.4 * jnp.abs(t)
                        s0 = s0 + lr * att_vals[f]
                    for f in range(4, 8):
                        t = (xi[e, pl.ds(f * 16, 16)]
                             + xj[e, pl.ds(f * 16, 16)])
                        lr = 0.6 * t + 0.4 * jnp.abs(t)
                        s1 = s1 + lr * att_vals[f]
                    av0 = jnp.where(lane == k, lane_sum(s0), av0)
                    av1 = jnp.where(lane == k, lane_sum(s1), av1)
                sl = pl.ds(g * 16, 16)
                ev0 = jnp.exp(av0)
                ev1 = jnp.exp(av1)
                a0[sl] = ev0
                a1[sl] = ev1
                for k in range(16):
                    tail = (jnp.where(lane == 0, ev0[k], 0.0)
                            + jnp.where(lane == 1, ev1[k], 0.0))
                    dbuf[g * 16 + k, pl.ds(0, 16)] = tail
                return carry4

            lax.fori_loop(0, CW // 16, alpha_body, 0)

            def weight_body(g, carry6):
                exv0 = a0[pl.ds(g * 16, 16)]
                exv1 = a1[pl.ds(g * 16, 16)]
                for k in range(16):
                    e = g * 16 + k
                    ex0 = exv0[k]
                    ex1 = exv1[k]
                    for f in range(4):
                        sl = pl.ds(f * 16, 16)
                        wbuf[e, sl] = xj[e, sl] * ex0
                    for f in range(4, 8):
                        sl = pl.ds(f * 16, 16)
                        wbuf[e, sl] = xj[e, sl] * ex1
                return carry6

            lax.fori_loop(0, CW // 16, weight_body, 0)
            pltpu.sync_copy(wbuf, acc.at[dst2_v.at[j]], add=True)
            pltpu.sync_copy(dbuf, dacc.at[dst2_v.at[j]], add=True)
            return carry3

        lax.fori_loop(0, NCH, chunk_body, 0)
        plsc.subcore_barrier()
        pltpu.sync_copy(acc.at[pl.ds(r0, ACC_R // N_TILES)],
                        out_hbm.at[b, pl.ds(r0, ACC_R // N_TILES)])
        pltpu.sync_copy(dacc.at[pl.ds(r0, ACC_R // N_TILES)],
                        dout_hbm.at[b, pl.ds(r0, ACC_R // N_TILES)])
        return carry

    lax.fori_loop(0, BPC, batch_body, 0)


@jax.jit
def _sc_gat(xl_flat, xr_flat, src_t, dstg_t, dst2_t, att2, zeros_acc):
    mesh = plsc.VectorSubcoreMesh(core_axis_name="c", subcore_axis_name="s")
    f = pl.kernel(
        _sc_gat_body,
        mesh=mesh,
        out_type=(
            jax.ShapeDtypeStruct((B, ACC_R, FH), jnp.float32),
            jax.ShapeDtypeStruct((B, ACC_R, FH), jnp.float32),
        ),
        scratch_types=[
            pltpu.VMEM((EPT,), jnp.int32),
            pltpu.VMEM((EPT,), jnp.int32),
            pltpu.VMEM((EPT,), jnp.int32),
            pltpu.VMEM((EPT,), jnp.int32),
            pltpu.VMEM((NCH, CW), jnp.int32),
            pltpu.VMEM((8, 16), jnp.float32),
            pltpu.VMEM((CW, FH), jnp.float32),
            pltpu.VMEM((CW, FH), jnp.float32),
            pltpu.VMEM((CW, FH), jnp.float32),
            pltpu.VMEM((CW, FH), jnp.float32),
            pltpu.VMEM((CW,), jnp.float32),
            pltpu.VMEM((CW,), jnp.float32),
            pltpu.VMEM_SHARED((ACC_R, FH), jnp.float32),
            pltpu.VMEM_SHARED((ACC_R, FH), jnp.float32),
            pltpu.SemaphoreType.DMA,
        ],
    )
    return f(xl_flat, xr_flat, src_t, dstg_t, dst2_t, att2, zeros_acc)


def kernel(X_input, edge_index, return_attention_weights, lin_l_w, lin_l_b,
           lin_r_w, lin_r_b, att, gat_bias, lin_w, lin_b, ff1_w, ff1_b,
           ff2_w, ff2_b, ln1_w, ln1_b, ln2_w, ln2_b):
    w_cat = jnp.concatenate([lin_l_w.T, lin_r_w.T], axis=1)
    b_cat = jnp.concatenate([lin_l_b, lin_r_b]).reshape(1, 2 * FH)
    xlr = _proj(X_input, w_cat, b_cat)
    xl_flat = xlr[:, :, :FH].reshape(B * N_G, FH)
    xr_flat = xlr[:, :, FH:].reshape(B * N_G, FH)

    loop = jnp.arange(N_G, dtype=jnp.int32)
    n_real = edge_index.shape[1] + N_G
    pad = E_PAD - n_real
    src = jnp.concatenate(
        [edge_index[0].astype(jnp.int32), loop, jnp.zeros(pad, jnp.int32)])
    dst = jnp.concatenate(
        [edge_index[1].astype(jnp.int32), loop,
         jnp.full((pad,), N_G, jnp.int32)])
    dstg = jnp.where(dst == N_G, 0, dst)
    src_t = src.reshape(N_TILES, EPT)
    dstg_t = dstg.reshape(N_TILES, EPT)
    dst2_t = dst.reshape(N_TILES, NCH, CW)
    att2 = att.reshape(8, 16)
    zeros_acc = jnp.zeros((ACC_R, FH), jnp.float32)
    acc, dout = _sc_gat(xl_flat, xr_flat, src_t, dstg_t, dst2_t, att2,
                        zeros_acc)

    x_gat_t = acc[:, :N_G, :].transpose(1, 0, 2)
    den_t = dout[:, :N_G, 0:2].transpose(1, 0, 2)
    x_in_t = X_input.transpose(1, 0, 2)
    out_t = _dense_block(x_gat_t, den_t, x_in_t, gat_bias, lin_w, lin_b,
                         ff1_w, ff1_b, ff2_w, ff2_b, ln1_w, ln1_b, ln2_w,
                         ln2_b)
    return out_t.transpose(1, 0, 2)


# speedup vs baseline: 1.3402x; 1.3402x over previous
import functools
import jax
import jax.numpy as jnp
from jax import lax
from jax.experimental import pallas as pl
from jax.experimental.pallas import tpu as pltpu
from jax.experimental.pallas import tpu_sc as plsc

N_G = 978
F = 64
H = 2
FH = 128
B = 16
D_FF = 100
NEG = 0.2
GB = 163
N_STEPS = N_G // GB


def _proj_body(x_ref, w_ref, b_ref, o_ref):
    o_ref[0] = (
        jnp.dot(x_ref[0], w_ref[...], preferred_element_type=jnp.float32)
        + b_ref[...]
    )


@jax.jit
def _proj(x, w_cat, b_cat):
    return pl.pallas_call(
        _proj_body,
        grid=(B,),
        in_specs=[
            pl.BlockSpec((1, N_G, F), lambda b: (b, 0, 0)),
            pl.BlockSpec((F, 2 * FH), lambda b: (0, 0)),
            pl.BlockSpec((1, 2 * FH), lambda b: (0, 0)),
        ],
        out_specs=pl.BlockSpec((1, N_G, 2 * FH), lambda b: (b, 0, 0)),
        out_shape=jax.ShapeDtypeStruct((B, N_G, 2 * FH), jnp.float32),
    )(x, w_cat, b_cat)


def _ln(x, w, b):
    mu = jnp.mean(x, axis=-1, keepdims=True)
    xc = x - mu
    var = jnp.mean(xc * xc, axis=-1, keepdims=True)
    return xc * jax.lax.rsqrt(var + 1e-5) * w + b


def _dense_body(xr_ref, den_ref, xin_ref, gatb_ref, linw_ref, linb_ref,
                f1w_ref, f1b_ref, f2w_ref, f2b_ref, ln1w_ref, ln1b_ref,
                ln2w_ref, ln2b_ref, o_ref):
    gatb = gatb_ref[...]
    ln1w = ln1w_ref[...]
    ln1b = ln1b_ref[...]
    ln2w = ln2w_ref[...]
    ln2b = ln2b_ref[...]

    def body(g, carry):
        xs = xr_ref[pl.ds(g, 1)][0]
        den = den_ref[pl.ds(g, 1)][0]
        d0 = den[:, 0:1] + 1e-16
        d1 = den[:, 1:2] + 1e-16
        xc = 0.5 * (xs[:, :F] / d0 + xs[:, F:FH] / d1) + gatb
        y = lax.dot_general(
            xc, linw_ref[pl.ds(g, 1)][0], (((1,), (1,)), ((), ())),
            preferred_element_type=jnp.float32) + linb_ref[0, pl.ds(g, 1)]
        x1 = _ln(xin_ref[pl.ds(g, 1)][0] + y, ln1w, ln1b)
        h = jnp.maximum(
            lax.dot_general(
                x1, f1w_ref[pl.ds(g, 1)][0], (((1,), (1,)), ((), ())),
                preferred_element_type=jnp.float32) + f1b_ref[0, pl.ds(g, 1)],
            0.0)
        y2 = lax.dot_general(
            h, f2w_ref[pl.ds(g, 1)][0], (((1,), (1,)), ((), ())),
            preferred_element_type=jnp.float32) + f2b_ref[0, pl.ds(g, 1)]
        o_ref[pl.ds(g, 1)] = _ln(x1 + y2, ln2w, ln2b)[None]
        return carry

    lax.fori_loop(0, GB, body, 0, unroll=4)


@jax.jit
def _dense_block(x_gat_t, den_t, x_in_t, gat_bias, lin_w, lin_b, ff1_w,
                 ff1_b, ff2_w, ff2_b, ln1_w, ln1_b, ln2_w, ln2_b):
    gb2 = lambda shape: pl.BlockSpec(shape, lambda i: (0, 0))
    return pl.pallas_call(
        _dense_body,
        grid=(N_STEPS,),
        in_specs=[
            pl.BlockSpec((GB, B, FH), lambda i: (i, 0, 0)),
            pl.BlockSpec((GB, B, 2), lambda i: (i, 0, 0)),
            pl.BlockSpec((GB, B, F), lambda i: (i, 0, 0)),
            gb2((1, F)),
            pl.BlockSpec((GB, F, F), lambda i: (i, 0, 0)),
            pl.BlockSpec((1, GB, F), lambda i: (i, 0, 0)),
            pl.BlockSpec((GB, D_FF, F), lambda i: (i, 0, 0)),
            pl.BlockSpec((1, GB, D_FF), lambda i: (i, 0, 0)),
            pl.BlockSpec((GB, F, D_FF), lambda i: (i, 0, 0)),
            pl.BlockSpec((1, GB, F), lambda i: (i, 0, 0)),
            gb2((1, F)),
            gb2((1, F)),
            gb2((1, F)),
            gb2((1, F)),
        ],
        out_specs=pl.BlockSpec((GB, B, F), lambda i: (i, 0, 0)),
        out_shape=jax.ShapeDtypeStruct((N_G, B, F), jnp.float32),
    )(x_gat_t, den_t, x_in_t, gat_bias.reshape(1, F), lin_w,
      lin_b.reshape(N_STEPS, GB, F), ff1_w, ff1_b.reshape(N_STEPS, GB, D_FF),
      ff2_w, ff2_b.reshape(N_STEPS, GB, F), ln1_w.reshape(1, F),
      ln1_b.reshape(1, F), ln2_w.reshape(1, F), ln2_b.reshape(1, F))


N_TILES = 16
CW = 96
NCH = 11
EPT = CW * NCH
E_PAD = EPT * N_TILES
ACC_R = 1024
BPC = B // 2


def _sc_gat_body(xl_hbm, xr_hbm, src_hbm, dstg_hbm, dst2_hbm, att_hbm,
                 zeros_hbm, out_hbm, dout_hbm, src_v, dstg_v, gsrc, gdst,
                 dst2_v, att_v, xi, xj, wbuf, dbuf, a0, a1, acc, dacc, sem):
    c = lax.axis_index("c")
    s = lax.axis_index("s")
    r0 = s * (ACC_R // N_TILES)

    pltpu.sync_copy(src_hbm.at[s], src_v)
    pltpu.sync_copy(dstg_hbm.at[s], dstg_v)
    pltpu.sync_copy(dst2_hbm.at[s], dst2_v)
    pltpu.sync_copy(att_hbm, att_v)
    att_vals = [att_v[f] for f in range(8)]
    lane = lax.iota(jnp.int32, 16)

    def lane_sum(v):
        for m in (8, 4, 2, 1):
            v = v + v.at[lane ^ m].get(mode="promise_in_bounds")
        return v

    pltpu.sync_copy(zeros_hbm.at[pl.ds(0, CW)], dbuf)

    def batch_body(bb, carry):
        b = c * BPC + bb
        pltpu.sync_copy(zeros_hbm.at[pl.ds(r0, ACC_R // N_TILES)],
                        acc.at[pl.ds(r0, ACC_R // N_TILES)])
        pltpu.sync_copy(zeros_hbm.at[pl.ds(r0, ACC_R // N_TILES)],
                        dacc.at[pl.ds(r0, ACC_R // N_TILES)])
        off = b * N_G

        def idx_body(k, carry2):
            sl = pl.ds(k * 16, 16)
            gsrc[sl] = src_v[sl] + off
            gdst[sl] = dstg_v[sl] + off
            return carry2

        lax.fori_loop(0, EPT // 16, idx_body, 0)
        plsc.subcore_barrier()

        def chunk_body(j, carry3):
            e0 = j * CW
            cp1 = pltpu.async_copy(xl_hbm.at[gsrc.at[pl.ds(e0, CW)]], xj, sem)
            cp2 = pltpu.async_copy(xr_hbm.at[gdst.at[pl.ds(e0, CW)]], xi, sem)
            cp1.wait()
            cp2.wait()

            def alpha_body(g, carry4):
                av0 = jnp.zeros((16,), jnp.float32)
                av1 = jnp.zeros((16,), jnp.float32)
                for k in range(16):
                    e = g * 16 + k
                    s0 = jnp.zeros((16,), jnp.float32)
                    s1 = jnp.zeros((16,), jnp.float32)
                    for f in range(4):
                        t = (xi[e, pl.ds(f * 16, 16)]
                             + xj[e, pl.ds(f * 16, 16)])
                        lr = 0.6 * t + 0.4 * jnp.abs(t)
                        s0 = s0 + lr * att_vals[f]
                    for f in range(4, 8):
                        t = (xi[e, pl.ds(f * 16, 16)]
                             + xj[e, pl.ds(f * 16, 16)])
                        lr = 0.6 * t + 0.4 * jnp.abs(t)
                        s1 = s1 + lr * att_vals[f]
                    av0 = jnp.where(lane == k, lane_sum(s0), av0)
                    av1 = jnp.where(lane == k, lane_sum(s1), av1)
                sl = pl.ds(g * 16, 16)
                ev0 = jnp.exp(av0)
                ev1 = jnp.exp(av1)
                a0[sl] = ev0
                a1[sl] = ev1
                for k in range(16):
                    tail = (jnp.where(lane == 0, ev0[k], 0.0)
                            + jnp.where(lane == 1, ev1[k], 0.0))
                    dbuf[g * 16 + k, pl.ds(0, 16)] = tail
                return carry4

            lax.fori_loop(0, CW // 16, alpha_body, 0)

            def weight_body(g, carry6):
                exv0 = a0[pl.ds(g * 16, 16)]
                exv1 = a1[pl.ds(g * 16, 16)]
                for k in range(16):
                    e = g * 16 + k
                    ex0 = exv0[k]
                    ex1 = exv1[k]
                    for f in range(4):
                        sl = pl.ds(f * 16, 16)
                        wbuf[e, sl] = xj[e, sl] * ex0
                    for f in range(4, 8):
                        sl = pl.ds(f * 16, 16)
                        wbuf[e, sl] = xj[e, sl] * ex1
                return carry6

            lax.fori_loop(0, CW // 16, weight_body, 0)
            pltpu.sync_copy(wbuf, acc.at[dst2_v.at[j]], add=True)
            pltpu.sync_copy(dbuf, dacc.at[dst2_v.at[j]], add=True)
            return carry3

        lax.fori_loop(0, NCH, chunk_body, 0)
        plsc.subcore_barrier()
        pltpu.sync_copy(acc.at[pl.ds(r0, ACC_R // N_TILES)],
                        out_hbm.at[b, pl.ds(r0, ACC_R // N_TILES)])
        pltpu.sync_copy(dacc.at[pl.ds(r0, ACC_R // N_TILES)],
                        dout_hbm.at[b, pl.ds(r0, ACC_R // N_TILES)])
        return carry

    lax.fori_loop(0, BPC, batch_body, 0)


@jax.jit
def _sc_gat(xl_flat, xr_flat, src_t, dstg_t, dst2_t, att2, zeros_acc):
    mesh = plsc.VectorSubcoreMesh(core_axis_name="c", subcore_axis_name="s")
    f = pl.kernel(
        _sc_gat_body,
        mesh=mesh,
        out_type=(
            jax.ShapeDtypeStruct((B, ACC_R, FH), jnp.float32),
            jax.ShapeDtypeStruct((B, ACC_R, FH), jnp.float32),
        ),
        scratch_types=[
            pltpu.VMEM((EPT,), jnp.int32),
            pltpu.VMEM((EPT,), jnp.int32),
            pltpu.VMEM((EPT,), jnp.int32),
            pltpu.VMEM((EPT,), jnp.int32),
            pltpu.VMEM((NCH, CW), jnp.int32),
            pltpu.VMEM((8, 16), jnp.float32),
            pltpu.VMEM((CW, FH), jnp.float32),
            pltpu.VMEM((CW, FH), jnp.float32),
            pltpu.VMEM((CW, FH), jnp.float32),
            pltpu.VMEM((CW, FH), jnp.float32),
            pltpu.VMEM((CW,), jnp.float32),
            pltpu.VMEM((CW,), jnp.float32),
            pltpu.VMEM_SHARED((ACC_R, FH), jnp.float32),
            pltpu.VMEM_SHARED((ACC_R, FH), jnp.float32),
            pltpu.SemaphoreType.DMA,
        ],
    )
    return f(xl_flat, xr_flat, src_t, dstg_t, dst2_t, att2, zeros_acc)


def kernel(X_input, edge_index, return_attention_weights, lin_l_w, lin_l_b,
           lin_r_w, lin_r_b, att, gat_bias, lin_w, lin_b, ff1_w, ff1_b,
           ff2_w, ff2_b, ln1_w, ln1_b, ln2_w, ln2_b):
    w_cat = jnp.concatenate([lin_l_w.T, lin_r_w.T], axis=1)
    b_cat = jnp.concatenate([lin_l_b, lin_r_b]).reshape(1, 2 * FH)
    xlr = _proj(X_input, w_cat, b_cat)
    xl_flat = xlr[:, :, :FH].reshape(B * N_G, FH)
    xr_flat = xlr[:, :, FH:].reshape(B * N_G, FH)

    loop = jnp.arange(N_G, dtype=jnp.int32)
    n_real = edge_index.shape[1] + N_G
    pad = E_PAD - n_real
    src = jnp.concatenate(
        [edge_index[0].astype(jnp.int32), loop, jnp.zeros(pad, jnp.int32)])
    dst = jnp.concatenate(
        [edge_index[1].astype(jnp.int32), loop,
         jnp.full((pad,), N_G, jnp.int32)])
    dstg = jnp.where(dst == N_G, 0, dst)
    src_t = src.reshape(N_TILES, EPT)
    dstg_t = dstg.reshape(N_TILES, EPT)
    dst2_t = dst.reshape(N_TILES, NCH, CW)
    att2 = att.reshape(8, 16)
    zeros_acc = jnp.zeros((ACC_R, FH), jnp.float32)
    acc, dout = _sc_gat(xl_flat, xr_flat, src_t, dstg_t, dst2_t, att2,
                        zeros_acc)

    x_gat_t = acc[:, :N_G, :].transpose(1, 0, 2)
    den_t = dout[:, :N_G, 0:2].transpose(1, 0, 2)
    x_in_t = X_input.transpose(1, 0, 2)
    out_t = _dense_block(x_gat_t, den_t, x_in_t, gat_bias, lin_w, lin_b,
                         ff1_w, ff1_b, ff2_w, ff2_b, ln1_w, ln1_b, ln2_w,
                         ln2_b)
    return out_t.transpose(1, 0, 2)


# speedup vs baseline: 1.4160x; 1.0566x over previous
import functools
import jax
import jax.numpy as jnp
from jax import lax
from jax.experimental import pallas as pl
from jax.experimental.pallas import tpu as pltpu
from jax.experimental.pallas import tpu_sc as plsc

N_G = 978
F = 64
H = 2
FH = 128
B = 16
D_FF = 100
NEG = 0.2
GB = 163
N_STEPS = N_G // GB


def _proj_body(x_ref, w_ref, b_ref, o_ref):
    o_ref[0] = (
        jnp.dot(x_ref[0], w_ref[...], preferred_element_type=jnp.float32)
        + b_ref[...]
    )


@jax.jit
def _proj(x, w_cat, b_cat):
    return pl.pallas_call(
        _proj_body,
        grid=(B,),
        in_specs=[
            pl.BlockSpec((1, N_G, F), lambda b: (b, 0, 0)),
            pl.BlockSpec((F, 2 * FH), lambda b: (0, 0)),
            pl.BlockSpec((1, 2 * FH), lambda b: (0, 0)),
        ],
        out_specs=pl.BlockSpec((1, N_G, 2 * FH), lambda b: (b, 0, 0)),
        out_shape=jax.ShapeDtypeStruct((B, N_G, 2 * FH), jnp.float32),
    )(x, w_cat, b_cat)


def _ln(x, w, b):
    mu = jnp.mean(x, axis=-1, keepdims=True)
    xc = x - mu
    var = jnp.mean(xc * xc, axis=-1, keepdims=True)
    return xc * jax.lax.rsqrt(var + 1e-5) * w + b


def _dense_body(xr_ref, den_ref, xin_ref, gatb_ref, linw_ref, linb_ref,
                f1w_ref, f1b_ref, f2w_ref, f2b_ref, ln1w_ref, ln1b_ref,
                ln2w_ref, ln2b_ref, o_ref):
    gatb = gatb_ref[...]
    ln1w = ln1w_ref[...]
    ln1b = ln1b_ref[...]
    ln2w = ln2w_ref[...]
    ln2b = ln2b_ref[...]

    def body(g, carry):
        xs = xr_ref[pl.ds(g, 1)][0]
        den = den_ref[pl.ds(g, 1)][0]
        d0 = den[:, 0:1] + 1e-16
        d1 = den[:, 1:2] + 1e-16
        xc = 0.5 * (xs[:, :F] / d0 + xs[:, F:FH] / d1) + gatb
        y = lax.dot_general(
            xc.astype(jnp.bfloat16),
            linw_ref[pl.ds(g, 1)][0].astype(jnp.bfloat16),
            (((1,), (1,)), ((), ())),
            preferred_element_type=jnp.float32) + linb_ref[0, pl.ds(g, 1)]
        x1 = _ln(xin_ref[pl.ds(g, 1)][0] + y, ln1w, ln1b)
        h = jnp.maximum(
            lax.dot_general(
                x1.astype(jnp.bfloat16),
                f1w_ref[pl.ds(g, 1)][0].astype(jnp.bfloat16),
                (((1,), (1,)), ((), ())),
                preferred_element_type=jnp.float32) + f1b_ref[0, pl.ds(g, 1)],
            0.0)
        y2 = lax.dot_general(
            h.astype(jnp.bfloat16),
            f2w_ref[pl.ds(g, 1)][0].astype(jnp.bfloat16),
            (((1,), (1,)), ((), ())),
            preferred_element_type=jnp.float32) + f2b_ref[0, pl.ds(g, 1)]
        o_ref[pl.ds(g, 1)] = _ln(x1 + y2, ln2w, ln2b)[None]
        return carry

    lax.fori_loop(0, GB, body, 0, unroll=8)


@jax.jit
def _dense_block(x_gat_t, den_t, x_in_t, gat_bias, lin_w, lin_b, ff1_w,
                 ff1_b, ff2_w, ff2_b, ln1_w, ln1_b, ln2_w, ln2_b):
    gb2 = lambda shape: pl.BlockSpec(shape, lambda i: (0, 0))
    return pl.pallas_call(
        _dense_body,
        grid=(N_STEPS,),
        in_specs=[
            pl.BlockSpec((GB, B, FH), lambda i: (i, 0, 0)),
            pl.BlockSpec((GB, B, 2), lambda i: (i, 0, 0)),
            pl.BlockSpec((GB, B, F), lambda i: (i, 0, 0)),
            gb2((1, F)),
            pl.BlockSpec((GB, F, F), lambda i: (i, 0, 0)),
            pl.BlockSpec((1, GB, F), lambda i: (i, 0, 0)),
            pl.BlockSpec((GB, D_FF, F), lambda i: (i, 0, 0)),
            pl.BlockSpec((1, GB, D_FF), lambda i: (i, 0, 0)),
            pl.BlockSpec((GB, F, D_FF), lambda i: (i, 0, 0)),
            pl.BlockSpec((1, GB, F), lambda i: (i, 0, 0)),
            gb2((1, F)),
            gb2((1, F)),
            gb2((1, F)),
            gb2((1, F)),
        ],
        out_specs=pl.BlockSpec((GB, B, F), lambda i: (i, 0, 0)),
        out_shape=jax.ShapeDtypeStruct((N_G, B, F), jnp.float32),
    )(x_gat_t, den_t, x_in_t, gat_bias.reshape(1, F), lin_w,
      lin_b.reshape(N_STEPS, GB, F), ff1_w, ff1_b.reshape(N_STEPS, GB, D_FF),
      ff2_w, ff2_b.reshape(N_STEPS, GB, F), ln1_w.reshape(1, F),
      ln1_b.reshape(1, F), ln2_w.reshape(1, F), ln2_b.reshape(1, F))


N_TILES = 16
CW = 96
NCH = 11
EPT = CW * NCH
E_PAD = EPT * N_TILES
ACC_R = 1024
BPC = B // 2


def _sc_gat_body(xl_hbm, xr_hbm, src_hbm, dstg_hbm, dst2_hbm, att_hbm,
                 zeros_hbm, out_hbm, dout_hbm, src_v, dstg_v, gsrc, gdst,
                 dst2_v, att_v, xi, xj, wbuf, dbuf, a0, a1, acc, dacc, sem):
    c = lax.axis_index("c")
    s = lax.axis_index("s")
    r0 = s * (ACC_R // N_TILES)

    pltpu.sync_copy(src_hbm.at[s], src_v)
    pltpu.sync_copy(dstg_hbm.at[s], dstg_v)
    pltpu.sync_copy(dst2_hbm.at[s], dst2_v)
    pltpu.sync_copy(att_hbm, att_v)
    att_vals = [att_v[f] for f in range(8)]
    lane = lax.iota(jnp.int32, 16)

    def lane_sum(v):
        for m in (8, 4, 2, 1):
            v = v + v.at[lane ^ m].get(mode="promise_in_bounds")
        return v

    pltpu.sync_copy(zeros_hbm.at[pl.ds(0, CW)], dbuf)

    def batch_body(bb, carry):
        b = c * BPC + bb
        pltpu.sync_copy(zeros_hbm.at[pl.ds(r0, ACC_R // N_TILES)],
                        acc.at[pl.ds(r0, ACC_R // N_TILES)])
        pltpu.sync_copy(zeros_hbm.at[pl.ds(r0, ACC_R // N_TILES)],
                        dacc.at[pl.ds(r0, ACC_R // N_TILES)])
        off = b * N_G

        def idx_body(k, carry2):
            sl = pl.ds(k * 16, 16)
            gsrc[sl] = src_v[sl] + off
            gdst[sl] = dstg_v[sl] + off
            return carry2

        lax.fori_loop(0, EPT // 16, idx_body, 0)
        plsc.subcore_barrier()

        def chunk_body(j, carry3):
            e0 = j * CW
            cp1 = pltpu.async_copy(xl_hbm.at[gsrc.at[pl.ds(e0, CW)]], xj, sem)
            cp2 = pltpu.async_copy(xr_hbm.at[gdst.at[pl.ds(e0, CW)]], xi, sem)
            cp1.wait()
            cp2.wait()

            def alpha_body(g, carry4):
                av0 = jnp.zeros((16,), jnp.float32)
                av1 = jnp.zeros((16,), jnp.float32)
                for k in range(16):
                    e = g * 16 + k
                    s0 = jnp.zeros((16,), jnp.float32)
                    s1 = jnp.zeros((16,), jnp.float32)
                    for f in range(4):
                        t = (xi[e, pl.ds(f * 16, 16)]
                             + xj[e, pl.ds(f * 16, 16)])
                        lr = 0.6 * t + 0.4 * jnp.abs(t)
                        s0 = s0 + lr * att_vals[f]
                    for f in range(4, 8):
                        t = (xi[e, pl.ds(f * 16, 16)]
                             + xj[e, pl.ds(f * 16, 16)])
                        lr = 0.6 * t + 0.4 * jnp.abs(t)
                        s1 = s1 + lr * att_vals[f]
                    av0 = jnp.where(lane == k, lane_sum(s0), av0)
                    av1 = jnp.where(lane == k, lane_sum(s1), av1)
                sl = pl.ds(g * 16, 16)
                ev0 = jnp.exp(av0)
                ev1 = jnp.exp(av1)
                a0[sl] = ev0
                a1[sl] = ev1
                for k in range(16):
                    tail = (jnp.where(lane == 0, ev0[k], 0.0)
                            + jnp.where(lane == 1, ev1[k], 0.0))
                    dbuf[g * 16 + k, pl.ds(0, 16)] = tail
                return carry4

            lax.fori_loop(0, CW // 16, alpha_body, 0)

            def weight_body(g, carry6):
                exv0 = a0[pl.ds(g * 16, 16)]
                exv1 = a1[pl.ds(g * 16, 16)]
                for k in range(16):
                    e = g * 16 + k
                    ex0 = exv0[k]
                    ex1 = exv1[k]
                    for f in range(4):
                        sl = pl.ds(f * 16, 16)
                        wbuf[e, sl] = xj[e, sl] * ex0
                    for f in range(4, 8):
                        sl = pl.ds(f * 16, 16)
                        wbuf[e, sl] = xj[e, sl] * ex1
                return carry6

            lax.fori_loop(0, CW // 16, weight_body, 0)
            pltpu.sync_copy(wbuf, acc.at[dst2_v.at[j]], add=True)
            pltpu.sync_copy(dbuf, dacc.at[dst2_v.at[j]], add=True)
            return carry3

        lax.fori_loop(0, NCH, chunk_body, 0)
        plsc.subcore_barrier()
        pltpu.sync_copy(acc.at[pl.ds(r0, ACC_R // N_TILES)],
                        out_hbm.at[b, pl.ds(r0, ACC_R // N_TILES)])
        pltpu.sync_copy(dacc.at[pl.ds(r0, ACC_R // N_TILES)],
                        dout_hbm.at[b, pl.ds(r0, ACC_R // N_TILES)])
        return carry

    lax.fori_loop(0, BPC, batch_body, 0)


@jax.jit
def _sc_gat(xl_flat, xr_flat, src_t, dstg_t, dst2_t, att2, zeros_acc):
    mesh = plsc.VectorSubcoreMesh(core_axis_name="c", subcore_axis_name="s")
    f = pl.kernel(
        _sc_gat_body,
        mesh=mesh,
        out_type=(
            jax.ShapeDtypeStruct((B, ACC_R, FH), jnp.float32),
            jax.ShapeDtypeStruct((B, ACC_R, FH), jnp.float32),
        ),
        scratch_types=[
            pltpu.VMEM((EPT,), jnp.int32),
            pltpu.VMEM((EPT,), jnp.int32),
            pltpu.VMEM((EPT,), jnp.int32),
            pltpu.VMEM((EPT,), jnp.int32),
            pltpu.VMEM((NCH, CW), jnp.int32),
            pltpu.VMEM((8, 16), jnp.float32),
            pltpu.VMEM((CW, FH), jnp.float32),
            pltpu.VMEM((CW, FH), jnp.float32),
            pltpu.VMEM((CW, FH), jnp.float32),
            pltpu.VMEM((CW, FH), jnp.float32),
            pltpu.VMEM((CW,), jnp.float32),
            pltpu.VMEM((CW,), jnp.float32),
            pltpu.VMEM_SHARED((ACC_R, FH), jnp.float32),
            pltpu.VMEM_SHARED((ACC_R, FH), jnp.float32),
            pltpu.SemaphoreType.DMA,
        ],
    )
    return f(xl_flat, xr_flat, src_t, dstg_t, dst2_t, att2, zeros_acc)


def kernel(X_input, edge_index, return_attention_weights, lin_l_w, lin_l_b,
           lin_r_w, lin_r_b, att, gat_bias, lin_w, lin_b, ff1_w, ff1_b,
           ff2_w, ff2_b, ln1_w, ln1_b, ln2_w, ln2_b):
    w_cat = jnp.concatenate([lin_l_w.T, lin_r_w.T], axis=1)
    b_cat = jnp.concatenate([lin_l_b, lin_r_b]).reshape(1, 2 * FH)
    xlr = _proj(X_input, w_cat, b_cat)
    xl_flat = xlr[:, :, :FH].reshape(B * N_G, FH)
    xr_flat = xlr[:, :, FH:].reshape(B * N_G, FH)

    loop = jnp.arange(N_G, dtype=jnp.int32)
    n_real = edge_index.shape[1] + N_G
    pad = E_PAD - n_real
    src = jnp.concatenate(
        [edge_index[0].astype(jnp.int32), loop, jnp.zeros(pad, jnp.int32)])
    dst = jnp.concatenate(
        [edge_index[1].astype(jnp.int32), loop,
         jnp.full((pad,), N_G, jnp.int32)])
    dstg = jnp.where(dst == N_G, 0, dst)
    src_t = src.reshape(N_TILES, EPT)
    dstg_t = dstg.reshape(N_TILES, EPT)
    dst2_t = dst.reshape(N_TILES, NCH, CW)
    att2 = att.reshape(8, 16)
    zeros_acc = jnp.zeros((ACC_R, FH), jnp.float32)
    acc, dout = _sc_gat(xl_flat, xr_flat, src_t, dstg_t, dst2_t, att2,
                        zeros_acc)

    x_gat_t = acc[:, :N_G, :].transpose(1, 0, 2)
    den_t = dout[:, :N_G, 0:2].transpose(1, 0, 2)
    x_in_t = X_input.transpose(1, 0, 2)
    out_t = _dense_block(x_gat_t, den_t, x_in_t, gat_bias, lin_w, lin_b,
                         ff1_w, ff1_b, ff2_w, ff2_b, ln1_w, ln1_b, ln2_w,
                         ln2_b)
    return out_t.transpose(1, 0, 2)


# speedup vs baseline: 1.6357x; 1.1551x over previous
import functools
import jax
import jax.numpy as jnp
from jax import lax
from jax.experimental import pallas as pl
from jax.experimental.pallas import tpu as pltpu
from jax.experimental.pallas import tpu_sc as plsc

N_G = 978
F = 64
H = 2
FH = 128
B = 16
D_FF = 100
NEG = 0.2
GB = 163
N_STEPS = N_G // GB


def _proj_body(x_ref, w_ref, b_ref, o_ref):
    o_ref[0] = (
        jnp.dot(x_ref[0], w_ref[...], preferred_element_type=jnp.float32)
        + b_ref[...]
    )


@jax.jit
def _proj(x, w_cat, b_cat):
    return pl.pallas_call(
        _proj_body,
        grid=(B,),
        in_specs=[
            pl.BlockSpec((1, N_G, F), lambda b: (b, 0, 0)),
            pl.BlockSpec((F, 2 * FH), lambda b: (0, 0)),
            pl.BlockSpec((1, 2 * FH), lambda b: (0, 0)),
        ],
        out_specs=pl.BlockSpec((1, N_G, 2 * FH), lambda b: (b, 0, 0)),
        out_shape=jax.ShapeDtypeStruct((B, N_G, 2 * FH), jnp.float32),
    )(x, w_cat, b_cat)


def _ln(x, w, b):
    mu = jnp.mean(x, axis=-1, keepdims=True)
    xc = x - mu
    var = jnp.mean(xc * xc, axis=-1, keepdims=True)
    return xc * jax.lax.rsqrt(var + 1e-5) * w + b


def _dense_body(xr_ref, den_ref, xin_ref, gatb_ref, linw_ref, linb_ref,
                f1w_ref, f1b_ref, f2w_ref, f2b_ref, ln1w_ref, ln1b_ref,
                ln2w_ref, ln2b_ref, o_ref):
    gatb = gatb_ref[...]
    ln1w = ln1w_ref[...]
    ln1b = ln1b_ref[...]
    ln2w = ln2w_ref[...]
    ln2b = ln2b_ref[...]

    def body(g, carry):
        xs = xr_ref[pl.ds(g, 1)][0]
        den = den_ref[pl.ds(g, 1)][0]
        d0 = den[:, 0:1] + 1e-16
        d1 = den[:, 1:2] + 1e-16
        xc = 0.5 * (xs[:, :F] / d0 + xs[:, F:FH] / d1) + gatb
        y = lax.dot_general(
            xc.astype(jnp.bfloat16),
            linw_ref[pl.ds(g, 1)][0].astype(jnp.bfloat16),
            (((1,), (1,)), ((), ())),
            preferred_element_type=jnp.float32) + linb_ref[0, pl.ds(g, 1)]
        x1 = _ln(xin_ref[pl.ds(g, 1)][0] + y, ln1w, ln1b)
        h = jnp.maximum(
            lax.dot_general(
                x1.astype(jnp.bfloat16),
                f1w_ref[pl.ds(g, 1)][0].astype(jnp.bfloat16),
                (((1,), (1,)), ((), ())),
                preferred_element_type=jnp.float32) + f1b_ref[0, pl.ds(g, 1)],
            0.0)
        y2 = lax.dot_general(
            h.astype(jnp.bfloat16),
            f2w_ref[pl.ds(g, 1)][0].astype(jnp.bfloat16),
            (((1,), (1,)), ((), ())),
            preferred_element_type=jnp.float32) + f2b_ref[0, pl.ds(g, 1)]
        o_ref[pl.ds(g, 1)] = _ln(x1 + y2, ln2w, ln2b)[None]
        return carry

    lax.fori_loop(0, GB, body, 0, unroll=8)


@jax.jit
def _dense_block(x_gat_t, den_t, x_in_t, gat_bias, lin_w, lin_b, ff1_w,
                 ff1_b, ff2_w, ff2_b, ln1_w, ln1_b, ln2_w, ln2_b):
    gb2 = lambda shape: pl.BlockSpec(shape, lambda i: (0, 0))
    return pl.pallas_call(
        _dense_body,
        grid=(N_STEPS,),
        in_specs=[
            pl.BlockSpec((GB, B, FH), lambda i: (i, 0, 0)),
            pl.BlockSpec((GB, B, 2), lambda i: (i, 0, 0)),
            pl.BlockSpec((GB, B, F), lambda i: (i, 0, 0)),
            gb2((1, F)),
            pl.BlockSpec((GB, F, F), lambda i: (i, 0, 0)),
            pl.BlockSpec((1, GB, F), lambda i: (i, 0, 0)),
            pl.BlockSpec((GB, D_FF, F), lambda i: (i, 0, 0)),
            pl.BlockSpec((1, GB, D_FF), lambda i: (i, 0, 0)),
            pl.BlockSpec((GB, F, D_FF), lambda i: (i, 0, 0)),
            pl.BlockSpec((1, GB, F), lambda i: (i, 0, 0)),
            gb2((1, F)),
            gb2((1, F)),
            gb2((1, F)),
            gb2((1, F)),
        ],
        out_specs=pl.BlockSpec((GB, B, F), lambda i: (i, 0, 0)),
        out_shape=jax.ShapeDtypeStruct((N_G, B, F), jnp.float32),
    )(x_gat_t, den_t, x_in_t, gat_bias.reshape(1, F), lin_w,
      lin_b.reshape(N_STEPS, GB, F), ff1_w, ff1_b.reshape(N_STEPS, GB, D_FF),
      ff2_w, ff2_b.reshape(N_STEPS, GB, F), ln1_w.reshape(1, F),
      ln1_b.reshape(1, F), ln2_w.reshape(1, F), ln2_b.reshape(1, F))


N_TILES = 16
CW = 96
NCH = 11
EPT = CW * NCH
E_PAD = EPT * N_TILES
ACC_R = 1024
BPC = B // 2


def _sc_gat_body(xl_hbm, xr_hbm, src_hbm, dstg_hbm, dst2_hbm, att_hbm,
                 zeros_hbm, out_hbm, dout_hbm, src_v, dstg_v, gsrc, gdst,
                 dst2_v, att_v, xi, xj, xi2, xj2, wbuf, dbuf, a0, a1, acc,
                 dacc, sem, sem2):
    c = lax.axis_index("c")
    s = lax.axis_index("s")
    r0 = s * (ACC_R // N_TILES)

    pltpu.sync_copy(src_hbm.at[s], src_v)
    pltpu.sync_copy(dstg_hbm.at[s], dstg_v)
    pltpu.sync_copy(dst2_hbm.at[s], dst2_v)
    pltpu.sync_copy(att_hbm, att_v)
    att_vals = [att_v[f] for f in range(8)]
    lane = lax.iota(jnp.int32, 16)

    def lane_sum(v):
        for m in (8, 4, 2, 1):
            v = v + v.at[lane ^ m].get(mode="promise_in_bounds")
        return v

    pltpu.sync_copy(zeros_hbm.at[pl.ds(0, CW)], dbuf)

    def batch_body(bb, carry):
        b = c * BPC + bb
        pltpu.sync_copy(zeros_hbm.at[pl.ds(r0, ACC_R // N_TILES)],
                        acc.at[pl.ds(r0, ACC_R // N_TILES)])
        pltpu.sync_copy(zeros_hbm.at[pl.ds(r0, ACC_R // N_TILES)],
                        dacc.at[pl.ds(r0, ACC_R // N_TILES)])
        off = b * N_G

        def idx_body(k, carry2):
            sl = pl.ds(k * 16, 16)
            gsrc[sl] = src_v[sl] + off
            gdst[sl] = dstg_v[sl] + off
            return carry2

        lax.fori_loop(0, EPT // 16, idx_body, 0)
        plsc.subcore_barrier()

        def gather_pair(j, xib, xjb, semb):
            e0 = j * CW
            pltpu.async_copy(xl_hbm.at[gsrc.at[pl.ds(e0, CW)]], xjb, semb)
            pltpu.async_copy(xr_hbm.at[gdst.at[pl.ds(e0, CW)]], xib, semb)

        def wait_pair(j, xib, xjb, semb):
            e0 = j * CW
            pltpu.make_async_copy(
                xl_hbm.at[gsrc.at[pl.ds(e0, CW)]], xjb, semb).wait()
            pltpu.make_async_copy(
                xr_hbm.at[gdst.at[pl.ds(e0, CW)]], xib, semb).wait()

        def compute_chunk(j, xi, xj):
            def alpha_body(g, carry4):
                av0 = jnp.zeros((16,), jnp.float32)
                av1 = jnp.zeros((16,), jnp.float32)
                for k in range(16):
                    e = g * 16 + k
                    s0 = jnp.zeros((16,), jnp.float32)
                    s1 = jnp.zeros((16,), jnp.float32)
                    for f in range(4):
                        t = (xi[e, pl.ds(f * 16, 16)]
                             + xj[e, pl.ds(f * 16, 16)])
                        lr = 0.6 * t + 0.4 * jnp.abs(t)
                        s0 = s0 + lr * att_vals[f]
                    for f in range(4, 8):
                        t = (xi[e, pl.ds(f * 16, 16)]
                             + xj[e, pl.ds(f * 16, 16)])
                        lr = 0.6 * t + 0.4 * jnp.abs(t)
                        s1 = s1 + lr * att_vals[f]
                    av0 = jnp.where(lane == k, lane_sum(s0), av0)
                    av1 = jnp.where(lane == k, lane_sum(s1), av1)
                sl = pl.ds(g * 16, 16)
                ev0 = jnp.exp(av0)
                ev1 = jnp.exp(av1)
                a0[sl] = ev0
                a1[sl] = ev1
                for k in range(16):
                    tail = (jnp.where(lane == 0, ev0[k], 0.0)
                            + jnp.where(lane == 1, ev1[k], 0.0))
                    dbuf[g * 16 + k, pl.ds(0, 16)] = tail
                return carry4

            lax.fori_loop(0, CW // 16, alpha_body, 0)

            def weight_body(g, carry6):
                exv0 = a0[pl.ds(g * 16, 16)]
                exv1 = a1[pl.ds(g * 16, 16)]
                for k in range(16):
                    e = g * 16 + k
                    ex0 = exv0[k]
                    ex1 = exv1[k]
                    for f in range(4):
                        sl = pl.ds(f * 16, 16)
                        wbuf[e, sl] = xj[e, sl] * ex0
                    for f in range(4, 8):
                        sl = pl.ds(f * 16, 16)
                        wbuf[e, sl] = xj[e, sl] * ex1
                return carry6

            lax.fori_loop(0, CW // 16, weight_body, 0)
            pltpu.sync_copy(wbuf, acc.at[dst2_v.at[j]], add=True)
            pltpu.sync_copy(dbuf, dacc.at[dst2_v.at[j]], add=True)

        gather_pair(0, xi, xj, sem)

        def pair_body(j2, carry3):
            jA = j2 * 2
            jB = jA + 1
            wait_pair(jA, xi, xj, sem)
            gather_pair(jB, xi2, xj2, sem2)
            compute_chunk(jA, xi, xj)
            wait_pair(jB, xi2, xj2, sem2)

            @pl.when(jA + 2 < NCH)
            def _prefetch_next():
                gather_pair(jA + 2, xi, xj, sem)

            compute_chunk(jB, xi2, xj2)
            return carry3

        lax.fori_loop(0, NCH // 2, pair_body, 0)
        wait_pair(NCH - 1, xi, xj, sem)
        compute_chunk(NCH - 1, xi, xj)
        plsc.subcore_barrier()
        pltpu.sync_copy(acc.at[pl.ds(r0, ACC_R // N_TILES)],
                        out_hbm.at[b, pl.ds(r0, ACC_R // N_TILES)])
        pltpu.sync_copy(dacc.at[pl.ds(r0, ACC_R // N_TILES)],
                        dout_hbm.at[b, pl.ds(r0, ACC_R // N_TILES)])
        return carry

    lax.fori_loop(0, BPC, batch_body, 0)


@jax.jit
def _sc_gat(xl_flat, xr_flat, src_t, dstg_t, dst2_t, att2, zeros_acc):
    mesh = plsc.VectorSubcoreMesh(core_axis_name="c", subcore_axis_name="s")
    f = pl.kernel(
        _sc_gat_body,
        mesh=mesh,
        out_type=(
            jax.ShapeDtypeStruct((B, ACC_R, FH), jnp.float32),
            jax.ShapeDtypeStruct((B, ACC_R, FH), jnp.float32),
        ),
        scratch_types=[
            pltpu.VMEM((EPT,), jnp.int32),
            pltpu.VMEM((EPT,), jnp.int32),
            pltpu.VMEM((EPT,), jnp.int32),
            pltpu.VMEM((EPT,), jnp.int32),
            pltpu.VMEM((NCH, CW), jnp.int32),
            pltpu.VMEM((8, 16), jnp.float32),
            pltpu.VMEM((CW, FH), jnp.float32),
            pltpu.VMEM((CW, FH), jnp.float32),
            pltpu.VMEM((CW, FH), jnp.float32),
            pltpu.VMEM((CW, FH), jnp.float32),
            pltpu.VMEM((CW, FH), jnp.float32),
            pltpu.VMEM((CW, FH), jnp.float32),
            pltpu.VMEM((CW,), jnp.float32),
            pltpu.VMEM((CW,), jnp.float32),
            pltpu.VMEM_SHARED((ACC_R, FH), jnp.float32),
            pltpu.VMEM_SHARED((ACC_R, FH), jnp.float32),
            pltpu.SemaphoreType.DMA,
            pltpu.SemaphoreType.DMA,
        ],
    )
    return f(xl_flat, xr_flat, src_t, dstg_t, dst2_t, att2, zeros_acc)


def kernel(X_input, edge_index, return_attention_weights, lin_l_w, lin_l_b,
           lin_r_w, lin_r_b, att, gat_bias, lin_w, lin_b, ff1_w, ff1_b,
           ff2_w, ff2_b, ln1_w, ln1_b, ln2_w, ln2_b):
    w_cat = jnp.concatenate([lin_l_w.T, lin_r_w.T], axis=1)
    b_cat = jnp.concatenate([lin_l_b, lin_r_b]).reshape(1, 2 * FH)
    xlr = _proj(X_input, w_cat, b_cat)
    xl_flat = xlr[:, :, :FH].reshape(B * N_G, FH)
    xr_flat = xlr[:, :, FH:].reshape(B * N_G, FH)

    loop = jnp.arange(N_G, dtype=jnp.int32)
    n_real = edge_index.shape[1] + N_G
    pad = E_PAD - n_real
    src = jnp.concatenate(
        [edge_index[0].astype(jnp.int32), loop, jnp.zeros(pad, jnp.int32)])
    dst = jnp.concatenate(
        [edge_index[1].astype(jnp.int32), loop,
         jnp.full((pad,), N_G, jnp.int32)])
    dstg = jnp.where(dst == N_G, 0, dst)
    src_t = src.reshape(N_TILES, EPT)
    dstg_t = dstg.reshape(N_TILES, EPT)
    dst2_t = dst.reshape(N_TILES, NCH, CW)
    att2 = att.reshape(8, 16)
    zeros_acc = jnp.zeros((ACC_R, FH), jnp.float32)
    acc, dout = _sc_gat(xl_flat, xr_flat, src_t, dstg_t, dst2_t, att2,
                        zeros_acc)

    x_gat_t = acc[:, :N_G, :].transpose(1, 0, 2)
    den_t = dout[:, :N_G, 0:2].transpose(1, 0, 2)
    x_in_t = X_input.transpose(1, 0, 2)
    out_t = _dense_block(x_gat_t, den_t, x_in_t, gat_bias, lin_w, lin_b,
                         ff1_w, ff1_b, ff2_w, ff2_b, ln1_w, ln1_b, ln2_w,
                         ln2_b)
    return out_t.transpose(1, 0, 2)


# speedup vs baseline: 1.6995x; 1.0390x over previous
import functools
import jax
import jax.numpy as jnp
from jax import lax
from jax.experimental import pallas as pl
from jax.experimental.pallas import tpu as pltpu
from jax.experimental.pallas import tpu_sc as plsc

N_G = 978
F = 64
H = 2
FH = 128
B = 16
D_FF = 100
NEG = 0.2
GB = 163
N_STEPS = N_G // GB


def _proj_body(x_ref, w_ref, b_ref, o_ref):
    o_ref[0] = (
        jnp.dot(x_ref[0], w_ref[...], preferred_element_type=jnp.float32)
        + b_ref[...]
    )


@jax.jit
def _proj(x, w_cat, b_cat):
    return pl.pallas_call(
        _proj_body,
        grid=(B,),
        in_specs=[
            pl.BlockSpec((1, N_G, F), lambda b: (b, 0, 0)),
            pl.BlockSpec((F, 2 * FH), lambda b: (0, 0)),
            pl.BlockSpec((1, 2 * FH), lambda b: (0, 0)),
        ],
        out_specs=pl.BlockSpec((1, N_G, 2 * FH), lambda b: (b, 0, 0)),
        out_shape=jax.ShapeDtypeStruct((B, N_G, 2 * FH), jnp.float32),
    )(x, w_cat, b_cat)


def _ln(x, w, b):
    mu = jnp.mean(x, axis=-1, keepdims=True)
    xc = x - mu
    var = jnp.mean(xc * xc, axis=-1, keepdims=True)
    return xc * jax.lax.rsqrt(var + 1e-5) * w + b


def _dense_body(xr_ref, den_ref, xin_ref, gatb_ref, linw_ref, linb_ref,
                f1w_ref, f1b_ref, f2w_ref, f2b_ref, ln1w_ref, ln1b_ref,
                ln2w_ref, ln2b_ref, o_ref):
    gatb = gatb_ref[...]
    ln1w = ln1w_ref[...]
    ln1b = ln1b_ref[...]
    ln2w = ln2w_ref[...]
    ln2b = ln2b_ref[...]

    def body(g, carry):
        xs = xr_ref[pl.ds(g, 1)][0]
        den = den_ref[pl.ds(g, 1)][0]
        d0 = den[:, 0:1] + 1e-16
        d1 = den[:, 1:2] + 1e-16
        xc = 0.5 * (xs[:, :F] / d0 + xs[:, F:FH] / d1) + gatb
        y = lax.dot_general(
            xc.astype(jnp.bfloat16),
            linw_ref[pl.ds(g, 1)][0].astype(jnp.bfloat16),
            (((1,), (1,)), ((), ())),
            preferred_element_type=jnp.float32) + linb_ref[0, pl.ds(g, 1)]
        x1 = _ln(xin_ref[pl.ds(g, 1)][0] + y, ln1w, ln1b)
        h = jnp.maximum(
            lax.dot_general(
                x1.astype(jnp.bfloat16),
                f1w_ref[pl.ds(g, 1)][0].astype(jnp.bfloat16),
                (((1,), (1,)), ((), ())),
                preferred_element_type=jnp.float32) + f1b_ref[0, pl.ds(g, 1)],
            0.0)
        y2 = lax.dot_general(
            h.astype(jnp.bfloat16),
            f2w_ref[pl.ds(g, 1)][0].astype(jnp.bfloat16),
            (((1,), (1,)), ((), ())),
            preferred_element_type=jnp.float32) + f2b_ref[0, pl.ds(g, 1)]
        o_ref[pl.ds(g, 1)] = _ln(x1 + y2, ln2w, ln2b)[None]
        return carry

    lax.fori_loop(0, GB, body, 0, unroll=16)


@jax.jit
def _dense_block(x_gat_t, den_t, x_in_t, gat_bias, lin_w, lin_b, ff1_w,
                 ff1_b, ff2_w, ff2_b, ln1_w, ln1_b, ln2_w, ln2_b):
    gb2 = lambda shape: pl.BlockSpec(shape, lambda i: (0, 0))
    return pl.pallas_call(
        _dense_body,
        grid=(N_STEPS,),
        in_specs=[
            pl.BlockSpec((GB, B, FH), lambda i: (i, 0, 0)),
            pl.BlockSpec((GB, B, 2), lambda i: (i, 0, 0)),
            pl.BlockSpec((GB, B, F), lambda i: (i, 0, 0)),
            gb2((1, F)),
            pl.BlockSpec((GB, F, F), lambda i: (i, 0, 0)),
            pl.BlockSpec((1, GB, F), lambda i: (i, 0, 0)),
            pl.BlockSpec((GB, D_FF, F), lambda i: (i, 0, 0)),
            pl.BlockSpec((1, GB, D_FF), lambda i: (i, 0, 0)),
            pl.BlockSpec((GB, F, D_FF), lambda i: (i, 0, 0)),
            pl.BlockSpec((1, GB, F), lambda i: (i, 0, 0)),
            gb2((1, F)),
            gb2((1, F)),
            gb2((1, F)),
            gb2((1, F)),
        ],
        out_specs=pl.BlockSpec((GB, B, F), lambda i: (i, 0, 0)),
        out_shape=jax.ShapeDtypeStruct((N_G, B, F), jnp.float32),
    )(x_gat_t, den_t, x_in_t, gat_bias.reshape(1, F), lin_w,
      lin_b.reshape(N_STEPS, GB, F), ff1_w, ff1_b.reshape(N_STEPS, GB, D_FF),
      ff2_w, ff2_b.reshape(N_STEPS, GB, F), ln1_w.reshape(1, F),
      ln1_b.reshape(1, F), ln2_w.reshape(1, F), ln2_b.reshape(1, F))


N_TILES = 16
CW = 96
NCH = 11
EPT = CW * NCH
E_PAD = EPT * N_TILES
ACC_R = 1024
BPC = B // 2


def _sc_gat_body(xl_hbm, xr_hbm, src_hbm, dstg_hbm, dst2_hbm, att_hbm,
                 zeros_hbm, out_hbm, dout_hbm, src_v, dstg_v, gsrc, gdst,
                 dst2_v, att_v, xi, xj, xi2, xj2, wbuf, dbuf, a0, a1, acc,
                 dacc, sem, sem2):
    c = lax.axis_index("c")
    s = lax.axis_index("s")
    r0 = s * (ACC_R // N_TILES)

    pltpu.sync_copy(src_hbm.at[s], src_v)
    pltpu.sync_copy(dstg_hbm.at[s], dstg_v)
    pltpu.sync_copy(dst2_hbm.at[s], dst2_v)
    pltpu.sync_copy(att_hbm, att_v)
    att_vals = [att_v[f] for f in range(8)]
    lane = lax.iota(jnp.int32, 16)

    def lane_sum(v):
        for m in (8, 4, 2, 1):
            v = v + v.at[lane ^ m].get(mode="promise_in_bounds")
        return v

    pltpu.sync_copy(zeros_hbm.at[pl.ds(0, CW)], dbuf)

    def batch_body(bb, carry):
        b = c * BPC + bb
        pltpu.sync_copy(zeros_hbm.at[pl.ds(r0, ACC_R // N_TILES)],
                        acc.at[pl.ds(r0, ACC_R // N_TILES)])
        pltpu.sync_copy(zeros_hbm.at[pl.ds(r0, ACC_R // N_TILES)],
                        dacc.at[pl.ds(r0, ACC_R // N_TILES)])
        off = b * N_G

        def idx_body(k, carry2):
            sl = pl.ds(k * 16, 16)
            gsrc[sl] = src_v[sl] + off
            gdst[sl] = dstg_v[sl] + off
            return carry2

        lax.fori_loop(0, EPT // 16, idx_body, 0)
        plsc.subcore_barrier()

        def gather_pair(j, xib, xjb, semb):
            e0 = j * CW
            pltpu.async_copy(xl_hbm.at[gsrc.at[pl.ds(e0, CW)]], xjb, semb)
            pltpu.async_copy(xr_hbm.at[gdst.at[pl.ds(e0, CW)]], xib, semb)

        def wait_pair(j, xib, xjb, semb):
            e0 = j * CW
            pltpu.make_async_copy(
                xl_hbm.at[gsrc.at[pl.ds(e0, CW)]], xjb, semb).wait()
            pltpu.make_async_copy(
                xr_hbm.at[gdst.at[pl.ds(e0, CW)]], xib, semb).wait()

        def compute_chunk(j, xi, xj):
            def alpha_body(g, carry4):
                av0 = jnp.zeros((16,), jnp.float32)
                av1 = jnp.zeros((16,), jnp.float32)
                for k in range(16):
                    e = g * 16 + k
                    s0 = jnp.zeros((16,), jnp.float32)
                    s1 = jnp.zeros((16,), jnp.float32)
                    for f in range(4):
                        t = (xi[e, pl.ds(f * 16, 16)]
                             + xj[e, pl.ds(f * 16, 16)])
                        lr = 0.6 * t + 0.4 * jnp.abs(t)
                        s0 = s0 + lr * att_vals[f]
                    for f in range(4, 8):
                        t = (xi[e, pl.ds(f * 16, 16)]
                             + xj[e, pl.ds(f * 16, 16)])
                        lr = 0.6 * t + 0.4 * jnp.abs(t)
                        s1 = s1 + lr * att_vals[f]
                    av0 = jnp.where(lane == k, lane_sum(s0), av0)
                    av1 = jnp.where(lane == k, lane_sum(s1), av1)
                sl = pl.ds(g * 16, 16)
                ev0 = jnp.exp(av0)
                ev1 = jnp.exp(av1)
                a0[sl] = ev0
                a1[sl] = ev1
                for k in range(16):
                    tail = (jnp.where(lane == 0, ev0[k], 0.0)
                            + jnp.where(lane == 1, ev1[k], 0.0))
                    dbuf[g * 16 + k, pl.ds(0, 16)] = tail
                return carry4

            lax.fori_loop(0, CW // 16, alpha_body, 0)

            def weight_body(g, carry6):
                exv0 = a0[pl.ds(g * 16, 16)]
                exv1 = a1[pl.ds(g * 16, 16)]
                for k in range(16):
                    e = g * 16 + k
                    ex0 = exv0[k]
                    ex1 = exv1[k]
                    for f in range(4):
                        sl = pl.ds(f * 16, 16)
                        wbuf[e, sl] = xj[e, sl] * ex0
                    for f in range(4, 8):
                        sl = pl.ds(f * 16, 16)
                        wbuf[e, sl] = xj[e, sl] * ex1
                return carry6

            lax.fori_loop(0, CW // 16, weight_body, 0)
            pltpu.sync_copy(wbuf, acc.at[dst2_v.at[j]], add=True)
            pltpu.sync_copy(dbuf, dacc.at[dst2_v.at[j]], add=True)

        gather_pair(0, xi, xj, sem)

        def pair_body(j2, carry3):
            jA = j2 * 2
            jB = jA + 1
            wait_pair(jA, xi, xj, sem)
            gather_pair(jB, xi2, xj2, sem2)
            compute_chunk(jA, xi, xj)
            wait_pair(jB, xi2, xj2, sem2)

            @pl.when(jA + 2 < NCH)
            def _prefetch_next():
                gather_pair(jA + 2, xi, xj, sem)

            compute_chunk(jB, xi2, xj2)
            return carry3

        lax.fori_loop(0, NCH // 2, pair_body, 0)
        wait_pair(NCH - 1, xi, xj, sem)
        compute_chunk(NCH - 1, xi, xj)
        plsc.subcore_barrier()
        pltpu.sync_copy(acc.at[pl.ds(r0, ACC_R // N_TILES)],
                        out_hbm.at[b, pl.ds(r0, ACC_R // N_TILES)])
        pltpu.sync_copy(dacc.at[pl.ds(r0, ACC_R // N_TILES)],
                        dout_hbm.at[b, pl.ds(r0, ACC_R // N_TILES)])
        return carry

    lax.fori_loop(0, BPC, batch_body, 0)


@jax.jit
def _sc_gat(xl_flat, xr_flat, src_t, dstg_t, dst2_t, att2, zeros_acc):
    mesh = plsc.VectorSubcoreMesh(core_axis_name="c", subcore_axis_name="s")
    f = pl.kernel(
        _sc_gat_body,
        mesh=mesh,
        out_type=(
            jax.ShapeDtypeStruct((B, ACC_R, FH), jnp.float32),
            jax.ShapeDtypeStruct((B, ACC_R, FH), jnp.float32),
        ),
        scratch_types=[
            pltpu.VMEM((EPT,), jnp.int32),
            pltpu.VMEM((EPT,), jnp.int32),
            pltpu.VMEM((EPT,), jnp.int32),
            pltpu.VMEM((EPT,), jnp.int32),
            pltpu.VMEM((NCH, CW), jnp.int32),
            pltpu.VMEM((8, 16), jnp.float32),
            pltpu.VMEM((CW, FH), jnp.float32),
            pltpu.VMEM((CW, FH), jnp.float32),
            pltpu.VMEM((CW, FH), jnp.float32),
            pltpu.VMEM((CW, FH), jnp.float32),
            pltpu.VMEM((CW, FH), jnp.float32),
            pltpu.VMEM((CW, FH), jnp.float32),
            pltpu.VMEM((CW,), jnp.float32),
            pltpu.VMEM((CW,), jnp.float32),
            pltpu.VMEM_SHARED((ACC_R, FH), jnp.float32),
            pltpu.VMEM_SHARED((ACC_R, FH), jnp.float32),
            pltpu.SemaphoreType.DMA,
            pltpu.SemaphoreType.DMA,
        ],
    )
    return f(xl_flat, xr_flat, src_t, dstg_t, dst2_t, att2, zeros_acc)


def kernel(X_input, edge_index, return_attention_weights, lin_l_w, lin_l_b,
           lin_r_w, lin_r_b, att, gat_bias, lin_w, lin_b, ff1_w, ff1_b,
           ff2_w, ff2_b, ln1_w, ln1_b, ln2_w, ln2_b):
    w_cat = jnp.concatenate([lin_l_w.T, lin_r_w.T], axis=1)
    b_cat = jnp.concatenate([lin_l_b, lin_r_b]).reshape(1, 2 * FH)
    xlr = _proj(X_input, w_cat, b_cat)
    xl_flat = xlr[:, :, :FH].reshape(B * N_G, FH)
    xr_flat = xlr[:, :, FH:].reshape(B * N_G, FH)

    loop = jnp.arange(N_G, dtype=jnp.int32)
    n_real = edge_index.shape[1] + N_G
    pad = E_PAD - n_real
    src = jnp.concatenate(
        [edge_index[0].astype(jnp.int32), loop, jnp.zeros(pad, jnp.int32)])
    dst = jnp.concatenate(
        [edge_index[1].astype(jnp.int32), loop,
         jnp.full((pad,), N_G, jnp.int32)])
    dstg = jnp.where(dst == N_G, 0, dst)
    src_t = src.reshape(N_TILES, EPT)
    dstg_t = dstg.reshape(N_TILES, EPT)
    dst2_t = dst.reshape(N_TILES, NCH, CW)
    att2 = att.reshape(8, 16)
    zeros_acc = jnp.zeros((ACC_R, FH), jnp.float32)
    acc, dout = _sc_gat(xl_flat, xr_flat, src_t, dstg_t, dst2_t, att2,
                        zeros_acc)

    x_gat_t = acc[:, :N_G, :].transpose(1, 0, 2)
    den_t = dout[:, :N_G, 0:2].transpose(1, 0, 2)
    x_in_t = X_input.transpose(1, 0, 2)
    out_t = _dense_block(x_gat_t, den_t, x_in_t, gat_bias, lin_w, lin_b,
                         ff1_w, ff1_b, ff2_w, ff2_b, ln1_w, ln1_b, ln2_w,
                         ln2_b)
    return out_t.transpose(1, 0, 2)
